# baseline (speedup 1.0000x reference)
import functools

import jax
import jax.numpy as jnp
from jax import lax
from jax.experimental import pallas as pl
from jax.experimental.pallas import tpu as pltpu
from jax.experimental.pallas import tpu_sc as plsc

_B = 16
_N = 16384
_NPOINT = 512
_L = 16
_NCHUNK = _N // _L


def _fps_kernel(xt_hbm, far0_hbm, out_hbm, x_v, dist_v, out_v, far0_v):
    cid = lax.axis_index("c")
    sid = lax.axis_index("s")
    wid = sid * 2 + cid

    @pl.when(wid < _B)
    def _():
        b = wid
        pltpu.sync_copy(xt_hbm.at[b], x_v)
        pltpu.sync_copy(far0_hbm, far0_v)
        bvec = jnp.full((_L,), b, dtype=jnp.int32)
        far_v = plsc.load_gather(far0_v, [bvec])

        big = jnp.full((_L,), 1e10, dtype=jnp.float32)

        @plsc.parallel_loop(0, _N, 8 * _L, unroll=2)
        def _init(off):
            for u in range(8):
                dist_v[pl.ds(off + u * _L, _L)] = big

        lane = lax.iota(jnp.int32, _L)
        mask0 = lane == 0
        _U = 4

        def step(t, far_v):
            tvec = jnp.full((_L,), t, dtype=jnp.int32)
            plsc.store_scatter(out_v, [tvec], far_v, mask=mask0)
            c0 = plsc.load_gather(x_v, [far_v])
            c1 = plsc.load_gather(x_v, [far_v + _N])
            c2 = plsc.load_gather(x_v, [far_v + 2 * _N])

            init = tuple(
                (jnp.full((_L,), -1.0, jnp.float32), jnp.zeros((_L,), jnp.int32))
                for _ in range(_U)
            )

            @plsc.parallel_loop(0, _N, _U * _L, unroll=4, carry=init)
            def accs(goff, accs):
                out = []
                for u in range(_U):
                    best, bidx = accs[u]
                    off = goff + u * _L
                    d0 = x_v[pl.ds(off, _L)] - c0
                    d1 = x_v[pl.ds(_N + off, _L)] - c1
                    d2 = x_v[pl.ds(2 * _N + off, _L)] - c2
                    d = (d0 * d0 + d1 * d1) + d2 * d2
                    nd = jnp.minimum(dist_v[pl.ds(off, _L)], d)
                    dist_v[pl.ds(off, _L)] = nd
                    pred = nd > best
                    out.append((
                        jnp.where(pred, nd, best),
                        jnp.where(pred, lane + off, bidx),
                    ))
                return tuple(out)

            best, bidx = accs[0]
            for u in range(1, _U):
                b2, i2 = accs[u]
                pred = (b2 > best) | ((b2 == best) & (i2 < bidx))
                best = jnp.where(pred, b2, best)
                bidx = jnp.where(pred, i2, bidx)
            m = jnp.max(best)
            cand = jnp.where(best == m, bidx, jnp.full((_L,), 2**30, jnp.int32))
            nf = jnp.min(cand)
            return jnp.full((_L,), nf, dtype=jnp.int32)

        lax.fori_loop(0, _NPOINT, step, far_v)
        pltpu.sync_copy(out_v, out_hbm.at[b])


@jax.jit
def kernel(x):
    xt = x[:, :, :3].transpose(0, 2, 1).reshape(_B, 3 * _N)
    fkey = jax.random.key(42)
    far0 = jax.random.randint(fkey, (_B,), 0, _N, dtype=jnp.int32)

    mesh = plsc.VectorSubcoreMesh(
        core_axis_name="c", subcore_axis_name="s", num_cores=2, num_subcores=16
    )
    fps = pl.kernel(
        _fps_kernel,
        out_type=jax.ShapeDtypeStruct((_B, _NPOINT), jnp.int32),
        mesh=mesh,
        compiler_params=pltpu.CompilerParams(needs_layout_passes=False),
        scratch_types=[
            pltpu.VMEM((3 * _N,), jnp.float32),
            pltpu.VMEM((_N,), jnp.float32),
            pltpu.VMEM((_NPOINT,), jnp.int32),
            pltpu.VMEM((_B,), jnp.int32),
        ],
    )
    out = fps(xt, far0)
    return out.astype(jnp.int64)


# speedup vs baseline: 1.4181x; 1.4181x over previous
import functools

import jax
import jax.numpy as jnp
from jax import lax
from jax.experimental import pallas as pl
from jax.experimental.pallas import tpu as pltpu
from jax.experimental.pallas import tpu_sc as plsc

_B = 16
_N = 16384
_NPOINT = 512
_L = 16
_NCHUNK = _N // _L


def _fps_kernel(xt_hbm, far0_hbm, out_hbm, x_v, dist_v, out_v, far0_v):
    cid = lax.axis_index("c")
    sid = lax.axis_index("s")
    wid = sid * 2 + cid

    @pl.when(wid < _B)
    def _():
        b = wid
        pltpu.sync_copy(xt_hbm.at[b], x_v)
        pltpu.sync_copy(far0_hbm, far0_v)
        bvec = jnp.full((_L,), b, dtype=jnp.int32)
        far_v = plsc.load_gather(far0_v, [bvec])

        big = jnp.full((_L,), 1e10, dtype=jnp.float32)

        @plsc.parallel_loop(0, _N, 8 * _L, unroll=2)
        def _init(off):
            for u in range(8):
                dist_v[pl.ds(off + u * _L, _L)] = big

        lane = lax.iota(jnp.int32, _L)
        mask0 = lane == 0
        _U = 8

        def step(t, far_v):
            tvec = jnp.full((_L,), t, dtype=jnp.int32)
            plsc.store_scatter(out_v, [tvec], far_v, mask=mask0)
            c0 = plsc.load_gather(x_v, [far_v])
            c1 = plsc.load_gather(x_v, [far_v + _N])
            c2 = plsc.load_gather(x_v, [far_v + 2 * _N])

            init = tuple(
                (jnp.full((_L,), -1.0, jnp.float32), jnp.zeros((_L,), jnp.int32))
                for _ in range(_U)
            )

            @plsc.parallel_loop(0, _N, _U * _L, unroll=1, carry=init)
            def accs(goff, accs):
                out = []
                for u in range(_U):
                    best, bidx = accs[u]
                    off = goff + u * _L
                    d0 = x_v[pl.ds(off, _L)] - c0
                    d1 = x_v[pl.ds(_N + off, _L)] - c1
                    d2 = x_v[pl.ds(2 * _N + off, _L)] - c2
                    d = (d0 * d0 + d1 * d1) + d2 * d2
                    nd = jnp.minimum(dist_v[pl.ds(off, _L)], d)
                    dist_v[pl.ds(off, _L)] = nd
                    pred = nd > best
                    out.append((
                        jnp.where(pred, nd, best),
                        jnp.where(pred, lane + off, bidx),
                    ))
                return tuple(out)

            best, bidx = accs[0]
            for u in range(1, _U):
                b2, i2 = accs[u]
                pred = (b2 > best) | ((b2 == best) & (i2 < bidx))
                best = jnp.where(pred, b2, best)
                bidx = jnp.where(pred, i2, bidx)
            m = jnp.max(best)
            cand = jnp.where(best == m, bidx, jnp.full((_L,), 2**30, jnp.int32))
            nf = jnp.min(cand)
            return jnp.full((_L,), nf, dtype=jnp.int32)

        lax.fori_loop(0, _NPOINT, step, far_v)
        pltpu.sync_copy(out_v, out_hbm.at[b])


@jax.jit
def kernel(x):
    xt = x[:, :, :3].transpose(0, 2, 1).reshape(_B, 3 * _N)
    fkey = jax.random.key(42)
    far0 = jax.random.randint(fkey, (_B,), 0, _N, dtype=jnp.int32)

    mesh = plsc.VectorSubcoreMesh(
        core_axis_name="c", subcore_axis_name="s", num_cores=2, num_subcores=16
    )
    fps = pl.kernel(
        _fps_kernel,
        out_type=jax.ShapeDtypeStruct((_B, _NPOINT), jnp.int32),
        mesh=mesh,
        compiler_params=pltpu.CompilerParams(needs_layout_passes=False),
        scratch_types=[
            pltpu.VMEM((3 * _N,), jnp.float32),
            pltpu.VMEM((_N,), jnp.float32),
            pltpu.VMEM((_NPOINT,), jnp.int32),
            pltpu.VMEM((_B,), jnp.int32),
        ],
    )
    out = fps(xt, far0)
    return out.astype(jnp.int64)


# speedup vs baseline: 1.5435x; 1.0884x over previous
import functools

import jax
import jax.numpy as jnp
from jax import lax
from jax.experimental import pallas as pl
from jax.experimental.pallas import tpu as pltpu
from jax.experimental.pallas import tpu_sc as plsc

_B = 16
_N = 16384
_NPOINT = 512
_L = 16
_NCHUNK = _N // _L


def _fps_kernel(xt_hbm, far0_hbm, out_hbm, x_v, dist_v, out_v, far0_v):
    cid = lax.axis_index("c")
    sid = lax.axis_index("s")
    wid = sid * 2 + cid

    @pl.when(wid < _B)
    def _():
        b = wid
        pltpu.sync_copy(xt_hbm.at[b], x_v)
        pltpu.sync_copy(far0_hbm, far0_v)
        bvec = jnp.full((_L,), b, dtype=jnp.int32)
        far_v = plsc.load_gather(far0_v, [bvec])

        big = jnp.full((_L,), 1e10, dtype=jnp.float32)

        @plsc.parallel_loop(0, _N, 8 * _L, unroll=2)
        def _init(off):
            for u in range(8):
                dist_v[pl.ds(off + u * _L, _L)] = big

        lane = lax.iota(jnp.int32, _L)
        mask0 = lane == 0
        _U = 8

        def step(t, far_v):
            tvec = jnp.full((_L,), t, dtype=jnp.int32)
            plsc.store_scatter(out_v, [tvec], far_v, mask=mask0)
            c0 = plsc.load_gather(x_v, [far_v])
            c1 = plsc.load_gather(x_v, [far_v + _N])
            c2 = plsc.load_gather(x_v, [far_v + 2 * _N])

            init = tuple(
                (jnp.full((_L,), -1.0, jnp.float32), jnp.zeros((_L,), jnp.int32))
                for _ in range(_U)
            )

            @plsc.parallel_loop(0, _N, _U * _L, unroll=1, carry=init)
            def accs(goff, accs):
                goff_v = jnp.full((_L,), goff, dtype=jnp.int32)
                out = []
                for u in range(_U):
                    best, bidx = accs[u]
                    off = goff + u * _L
                    d0 = x_v[pl.ds(off, _L)] - c0
                    d1 = x_v[pl.ds(_N + off, _L)] - c1
                    d2 = x_v[pl.ds(2 * _N + off, _L)] - c2
                    d = (d0 * d0 + d1 * d1) + d2 * d2
                    nd = jnp.minimum(dist_v[pl.ds(off, _L)], d)
                    dist_v[pl.ds(off, _L)] = nd
                    pred = nd > best
                    out.append((
                        jnp.where(pred, nd, best),
                        jnp.where(pred, goff_v, bidx),
                    ))
                return tuple(out)

            accs = tuple(
                (bu, iu + (lane + u * _L)) for u, (bu, iu) in enumerate(accs)
            )
            best, bidx = accs[0]
            for u in range(1, _U):
                b2, i2 = accs[u]
                pred = (b2 > best) | ((b2 == best) & (i2 < bidx))
                best = jnp.where(pred, b2, best)
                bidx = jnp.where(pred, i2, bidx)
            m = jnp.max(best)
            cand = jnp.where(best == m, bidx, jnp.full((_L,), 2**30, jnp.int32))
            nf = jnp.min(cand)
            return jnp.full((_L,), nf, dtype=jnp.int32)

        lax.fori_loop(0, _NPOINT, step, far_v)
        pltpu.sync_copy(out_v, out_hbm.at[b])


@jax.jit
def kernel(x):
    xt = x[:, :, :3].transpose(0, 2, 1).reshape(_B, 3 * _N)
    fkey = jax.random.key(42)
    far0 = jax.random.randint(fkey, (_B,), 0, _N, dtype=jnp.int32)

    mesh = plsc.VectorSubcoreMesh(
        core_axis_name="c", subcore_axis_name="s", num_cores=2, num_subcores=16
    )
    fps = pl.kernel(
        _fps_kernel,
        out_type=jax.ShapeDtypeStruct((_B, _NPOINT), jnp.int32),
        mesh=mesh,
        compiler_params=pltpu.CompilerParams(needs_layout_passes=False),
        scratch_types=[
            pltpu.VMEM((3 * _N,), jnp.float32),
            pltpu.VMEM((_N,), jnp.float32),
            pltpu.VMEM((_NPOINT,), jnp.int32),
            pltpu.VMEM((_B,), jnp.int32),
        ],
    )
    out = fps(xt, far0)
    return out.astype(jnp.int64)


# speedup vs baseline: 1.5435x; 1.0000x over previous
import jax
import jax.numpy as jnp
from jax import lax
from jax.experimental import pallas as pl
from jax.experimental.pallas import tpu as pltpu
from jax.experimental.pallas import tpu_sc as plsc

_B = 16
_N = 16384
_NPOINT = 512
_L = 16
_NCHUNK = _N // _L


def _fps_kernel(xt_hbm, far0_hbm, out_hbm, x_v, dist_v, out_v, far0_v):
    cid = lax.axis_index("c")
    sid = lax.axis_index("s")
    wid = sid * 2 + cid

    @pl.when(wid < _B)
    def _():
        b = wid
        pltpu.sync_copy(xt_hbm.at[b], x_v)
        pltpu.sync_copy(far0_hbm, far0_v)
        bvec = jnp.full((_L,), b, dtype=jnp.int32)
        far_v = plsc.load_gather(far0_v, [bvec])

        big = jnp.full((_L,), 1e10, dtype=jnp.float32)

        @plsc.parallel_loop(0, _N, 8 * _L, unroll=2)
        def _init(off):
            for u in range(8):
                dist_v[pl.ds(off + u * _L, _L)] = big

        lane = lax.iota(jnp.int32, _L)
        mask0 = lane == 0
        _U = 8

        def step(t, far_v):
            tvec = jnp.full((_L,), t, dtype=jnp.int32)
            plsc.store_scatter(out_v, [tvec], far_v, mask=mask0)
            c0 = plsc.load_gather(x_v, [far_v])
            c1 = plsc.load_gather(x_v, [far_v + _N])
            c2 = plsc.load_gather(x_v, [far_v + 2 * _N])

            init = tuple(
                (jnp.full((_L,), -1.0, jnp.float32), jnp.zeros((_L,), jnp.int32))
                for _ in range(_U)
            )

            @plsc.parallel_loop(0, _N, _U * _L, unroll=1, carry=init)
            def accs(goff, accs):
                goff_v = jnp.full((_L,), goff, dtype=jnp.int32)
                out = []
                for u in range(_U):
                    best, bidx = accs[u]
                    off = goff + u * _L
                    d0 = x_v[pl.ds(off, _L)] - c0
                    d1 = x_v[pl.ds(_N + off, _L)] - c1
                    d2 = x_v[pl.ds(2 * _N + off, _L)] - c2
                    d = (d0 * d0 + d1 * d1) + d2 * d2
                    nd = jnp.minimum(dist_v[pl.ds(off, _L)], d)
                    dist_v[pl.ds(off, _L)] = nd
                    pred = nd > best
                    out.append((
                        jnp.where(pred, nd, best),
                        jnp.where(pred, goff_v, bidx),
                    ))
                return tuple(out)

            accs = tuple(
                (bu, iu + (lane + u * _L)) for u, (bu, iu) in enumerate(accs)
            )
            best, bidx = accs[0]
            for u in range(1, _U):
                b2, i2 = accs[u]
                pred = (b2 > best) | ((b2 == best) & (i2 < bidx))
                best = jnp.where(pred, b2, best)
                bidx = jnp.where(pred, i2, bidx)
            m = jnp.max(best)
            cand = jnp.where(best == m, bidx, jnp.full((_L,), 2**30, jnp.int32))
            nf = jnp.min(cand)
            return jnp.full((_L,), nf, dtype=jnp.int32)

        lax.fori_loop(0, _NPOINT, step, far_v)
        pltpu.sync_copy(out_v, out_hbm.at[b])


@jax.jit
def kernel(x):
    xt = x[:, :, :3].transpose(0, 2, 1).reshape(_B, 3 * _N)
    fkey = jax.random.key(42)
    far0 = jax.random.randint(fkey, (_B,), 0, _N, dtype=jnp.int32)

    mesh = plsc.VectorSubcoreMesh(
        core_axis_name="c", subcore_axis_name="s", num_cores=2, num_subcores=16
    )
    fps = pl.kernel(
        _fps_kernel,
        out_type=jax.ShapeDtypeStruct((_B, _NPOINT), jnp.int32),
        mesh=mesh,
        compiler_params=pltpu.CompilerParams(needs_layout_passes=False),
        scratch_types=[
            pltpu.VMEM((3 * _N,), jnp.float32),
            pltpu.VMEM((_N,), jnp.float32),
            pltpu.VMEM((_NPOINT,), jnp.int32),
            pltpu.VMEM((_B,), jnp.int32),
        ],
    )
    out = fps(xt, far0)
    return out.astype(jnp.int64)
